# baseline (speedup 1.0000x reference)
import functools

import jax
import jax.numpy as jnp
from jax import lax
from jax.experimental import pallas as pl
from jax.experimental.pallas import tpu as pltpu
from jax.experimental.pallas import tpu_sc as plsc

D0, N, D2 = 8, 100000, 64
B, L = 1024, 50
R = D0 * B * L
NW = 32
RPW = R // NW
IDX_W = 128
IDX_ROWS = RPW // IDX_W
CHUNK = 512
NG = RPW // CHUNK
GPC = CHUNK // IDX_W
Q = NW // D0


def _body(table_hbm, ind_hbm, out_hbm, idx_v, rows_v, gsem):
    wid = lax.axis_index("s") * 2 + lax.axis_index("c")
    d0 = wid // Q
    q = wid % Q

    pltpu.sync_copy(ind_hbm.at[pl.ds(pl.multiple_of(q * RPW, RPW), RPW)], idx_v)

    off_v = jnp.full((16,), d0 * N, dtype=jnp.int32)

    def _add_vec(i, carry):
        sl = pl.ds(pl.multiple_of(i * 16, 16), 16)
        idx_v[sl] = idx_v[sl] + off_v
        return carry

    lax.fori_loop(0, RPW // 16, _add_vec, 0)

    for g in range(NG):
        handles = []
        for j in range(GPC):
            row = g * GPC + j
            handles.append(
                pltpu.async_copy(
                    table_hbm.at[idx_v.at[pl.ds(row * IDX_W, IDX_W)]],
                    rows_v.at[pl.ds(j * IDX_W, IDX_W)],
                    gsem,
                )
            )
        for h in handles:
            h.wait()
        base = wid * RPW + g * CHUNK
        pltpu.sync_copy(rows_v, out_hbm.at[pl.ds(base, CHUNK)])


_gather = functools.partial(
    pl.kernel,
    mesh=plsc.VectorSubcoreMesh(core_axis_name="c", subcore_axis_name="s"),
    out_type=jax.ShapeDtypeStruct((R, D2), jnp.float32),
    scratch_types=[
        pltpu.VMEM((RPW,), jnp.int32),
        pltpu.VMEM((CHUNK, D2), jnp.float32),
        pltpu.SemaphoreType.DMA,
    ],
    compiler_params=pltpu.CompilerParams(use_tc_tiling_on_sc=False),
)(_body)


@jax.jit
def kernel(a, ind):
    table = a.reshape(D0 * N, D2)
    ind2 = ind.reshape(-1).astype(jnp.int32)
    out = _gather(table, ind2)
    return out.reshape(D0, B, L, D2)
